# baseline (speedup 1.0000x reference)
import functools

import jax
import jax.numpy as jnp
from jax import lax
from jax.experimental import pallas as pl
from jax.experimental.pallas import tpu as pltpu
from jax.experimental.pallas import tpu_sc as plsc

_NC, _NS, _L = 2, 16, 16
_NW = _NC * _NS

_CH = 4096


@functools.lru_cache(maxsize=None)
def _build(S, B, D, VS):
    NCHK = B // _CH
    mesh = plsc.VectorSubcoreMesh(core_axis_name="c", subcore_axis_name="s")

    @functools.partial(
        pl.kernel,
        mesh=mesh,
        out_type=jax.ShapeDtypeStruct((S, D, B), jnp.float32),
        compiler_params=pltpu.CompilerParams(
            use_tc_tiling_on_sc=False, needs_layout_passes=False
        ),
        scratch_types=[
            pltpu.VMEM((VS,), jnp.float32),
            pltpu.VMEM((2, _CH), jnp.int32),
            pltpu.VMEM((2, _CH), jnp.float32),
            pltpu.SemaphoreType.DMA,
            pltpu.SemaphoreType.DMA,
        ],
    )
    def k(table_hbm, in_hbm, out_hbm, seg_v, idx_v, res_v, isem, wsem):
        d = lax.axis_index("s") * _NC + lax.axis_index("c")

        def idx_load(s, c, b):
            pltpu.async_copy(
                in_hbm.at[s, pl.ds(c * _CH, _CH)], idx_v.at[b], isem
            )

        def wait_idx(s, c, b):
            pltpu.make_async_copy(
                in_hbm.at[s, pl.ds(c * _CH, _CH)], idx_v.at[b], isem
            ).wait()

        def out_write(s, c, b):
            pltpu.async_copy(
                res_v.at[b], out_hbm.at[s, d, pl.ds(c * _CH, _CH)], wsem
            )

        def wait_write(s, c, b):
            pltpu.make_async_copy(
                res_v.at[b], out_hbm.at[s, d, pl.ds(c * _CH, _CH)], wsem
            ).wait()

        idx_load(0, 0, 0)

        def slot_body(s, carry):
            pltpu.sync_copy(table_hbm.at[d, pl.ds(s * VS, VS)], seg_v)

            def chunk_body(c, carry1):
                b = lax.rem(c, 2)
                wait_idx(s, c, b)

                @pl.when(c + 1 < NCHK)
                def _():
                    idx_load(s, c + 1, 1 - b)

                @pl.when(jnp.logical_and(c + 1 == NCHK, s + 1 < S))
                def _():
                    idx_load(s + 1, 0, 1 - b)

                @pl.when(jnp.logical_or(s > 0, c >= 2))
                def _():
                    cc = lax.rem(c + 2, NCHK)
                    ss = s - lax.select(c >= 2, 0, 1)
                    wait_write(ss, lax.select(c >= 2, c - 2, cc), b)

                def gather_body(j, carry2):
                    sl = pl.ds(j * _L, _L)
                    res_v[b, sl] = plsc.load_gather(seg_v, [idx_v[b, sl]])
                    return carry2

                lax.fori_loop(0, _CH // _L, gather_body, 0)
                out_write(s, c, b)
                return carry1

            lax.fori_loop(0, NCHK, chunk_body, 0)
            return carry

        lax.fori_loop(0, S, slot_body, 0)
        for c in (NCHK - 2, NCHK - 1):
            wait_write(S - 1, c, c % 2)

    return k


def kernel(inputs, table):
    B, S = inputs.shape
    V, D = table.shape
    VS = V // S
    k = _build(S, B, D, VS)
    out = k(table.T, inputs.T)
    return out.transpose(2, 0, 1)


# speedup vs baseline: 4.4177x; 4.4177x over previous
import functools

import jax
import jax.numpy as jnp
from jax import lax
from jax.experimental import pallas as pl
from jax.experimental.pallas import tpu as pltpu
from jax.experimental.pallas import tpu_sc as plsc

_NC, _NS, _L = 2, 16, 16
_NW = _NC * _NS

_GR = 128
_NBUF = 4
_K = 2


@functools.lru_cache(maxsize=None)
def _build(S, B, D, VS):
    N = B * S
    RPW = N // (_NW * _GR)
    PAT = S * _L
    NV = RPW * (_GR // _L)
    assert N % (_NW * _GR) == 0 and RPW % _NBUF == 0

    mesh = plsc.VectorSubcoreMesh(core_axis_name="c", subcore_axis_name="s")

    @functools.partial(
        pl.kernel,
        mesh=mesh,
        out_type=jax.ShapeDtypeStruct((N, D), jnp.float32),
        compiler_params=pltpu.CompilerParams(
            use_tc_tiling_on_sc=False, needs_layout_passes=False
        ),
        scratch_types=[
            pltpu.VMEM((RPW, _GR), jnp.int32),
            pltpu.VMEM((PAT,), jnp.int32),
            pltpu.VMEM((_NBUF, _GR, D), jnp.float32),
        ]
        + [pltpu.SemaphoreType.DMA] * (2 * _NBUF),
    )
    def k(table_hbm, in_hbm, pat_hbm, out_hbm, idx_v, pat_v, rows_v, *sems):
        gsem, wsem = sems[:_NBUF], sems[_NBUF:]
        wid = lax.axis_index("s") * _NC + lax.axis_index("c")
        c0 = wid * RPW

        pltpu.sync_copy(in_hbm.at[pl.ds(c0, RPW)], idx_v)
        pltpu.sync_copy(pat_hbm, pat_v)

        def fuse(j, carry):
            r = lax.div(j, _GR // _L)
            o = lax.rem(j, _GR // _L) * _L
            q = lax.rem(j, S) * _L
            idx_v[r, pl.ds(o, _L)] = idx_v[r, pl.ds(o, _L)] + pat_v[pl.ds(q, _L)]
            return carry

        lax.fori_loop(0, NV, fuse, 0)

        def gstart(c, b):
            pltpu.async_copy(table_hbm.at[idx_v.at[c]], rows_v.at[b], gsem[b])

        def gwait(c, b):
            pltpu.make_async_copy(
                table_hbm.at[idx_v.at[c]], rows_v.at[b], gsem[b]
            ).wait()

        def wstart(c, b):
            pltpu.async_copy(
                rows_v.at[b], out_hbm.at[pl.ds((c0 + c) * _GR, _GR)], wsem[b]
            )

        def wwait(c, b):
            pltpu.make_async_copy(
                rows_v.at[b], out_hbm.at[pl.ds((c0 + c) * _GR, _GR)], wsem[b]
            ).wait()

        for g in range(_K):
            gstart(g, g)

        def group(o, carry):
            base = o * _NBUF
            for b in range(_NBUF):
                c = base + b
                gwait(c, b)
                wstart(c, b)
                g = c + _K
                bg = (b + _K) % _NBUF

                @pl.when(jnp.logical_and(g < RPW, g >= _NBUF))
                def _():
                    wwait(g - _NBUF, bg)

                @pl.when(g < RPW)
                def _():
                    gstart(g, bg)

            return carry

        lax.fori_loop(0, RPW // _NBUF, group, 0)
        for b in range(_NBUF):
            wwait(RPW - _NBUF + b, b)

    return k


def kernel(inputs, table):
    B, S = inputs.shape
    V, D = table.shape
    VS = V // S
    k = _build(S, B, D, VS)
    idx2d = inputs.reshape(-1, _GR)
    pat = jnp.tile(jnp.arange(S, dtype=jnp.int32) * VS, _L)
    out = k(table, idx2d, pat)
    return out.reshape(B, S, D)
